# baseline (speedup 1.0000x reference)
import functools

import jax
import jax.numpy as jnp
from jax import lax
from jax.experimental import pallas as pl
from jax.experimental.pallas import tpu as pltpu
from jax.experimental.pallas import tpu_sc as plsc

B = 16384
D = 128
NC = 2
NS = 16
NW = NC * NS
BPW = B // NW
CHUNK = 128
NCHUNK = BPW // CHUNK
LANES = 16


def _gmf_body(ut_hbm, it_hbm, ui_hbm, ii_hbm, out_hbm,
              ui_v, ii_v, u_buf, i_buf,
              sem_g0, sem_g1, sem_o0, sem_o1):
    sem_g = (sem_g0, sem_g1)
    sem_o = (sem_o0, sem_o1)
    wid = lax.axis_index("s") * NC + lax.axis_index("c")
    base = wid * BPW

    for j in range(NCHUNK):
        pltpu.sync_copy(ui_hbm.at[pl.ds(base + j * CHUNK, CHUNK)], ui_v.at[j])
        pltpu.sync_copy(ii_hbm.at[pl.ds(base + j * CHUNK, CHUNK)], ii_v.at[j])

    def gathers(j, s):
        cu = pltpu.async_copy(ut_hbm.at[ui_v.at[j]], u_buf.at[s], sem_g[s])
        ci = pltpu.async_copy(it_hbm.at[ii_v.at[j]], i_buf.at[s], sem_g[s])
        return cu, ci

    pend_g = gathers(0, 0)
    pend_o = None
    for j in range(NCHUNK):
        s = j % 2
        if pend_o is not None:
            pend_o.wait()
            pend_o = None
        if j + 1 < NCHUNK:
            next_g = gathers(j + 1, 1 - s)
        else:
            next_g = None
        pend_g[0].wait()
        pend_g[1].wait()
        pend_g = next_g

        @plsc.parallel_loop(0, CHUNK, unroll=2)
        def row_body(r):
            for g in range(D // LANES):
                sl = pl.ds(g * LANES, LANES)
                u_buf[s, r, sl] = u_buf[s, r, sl] * i_buf[s, r, sl]
        pend_o = pltpu.async_copy(
            u_buf.at[s], out_hbm.at[pl.ds(base + j * CHUNK, CHUNK)], sem_o[s])
    pend_o.wait()


@functools.partial(jax.jit, static_argnames=())
def _gmf(user_table, item_table, user_indices, item_indices):
    mesh = plsc.VectorSubcoreMesh(core_axis_name="c", subcore_axis_name="s")
    call = pl.kernel(
        _gmf_body,
        mesh=mesh,
        out_type=jax.ShapeDtypeStruct((B, D), jnp.float32),
        scratch_types=[
            pltpu.VMEM((NCHUNK, CHUNK), jnp.int32),
            pltpu.VMEM((NCHUNK, CHUNK), jnp.int32),
            pltpu.VMEM((2, CHUNK, D), jnp.float32),
            pltpu.VMEM((2, CHUNK, D), jnp.float32),
            pltpu.SemaphoreType.DMA,
            pltpu.SemaphoreType.DMA,
            pltpu.SemaphoreType.DMA,
            pltpu.SemaphoreType.DMA,
        ],
    )
    return call(user_table, item_table, user_indices, item_indices)


def kernel(user_indices, item_indices, user_table, item_table):
    return _gmf(user_table, item_table,
                user_indices.astype(jnp.int32), item_indices.astype(jnp.int32))


# speedup vs baseline: 1.1238x; 1.1238x over previous
import functools

import jax
import jax.numpy as jnp
from jax import lax
from jax.experimental import pallas as pl
from jax.experimental.pallas import tpu as pltpu
from jax.experimental.pallas import tpu_sc as plsc

B = 16384
D = 128
NC = 2
NS = 16
NW = NC * NS
BPW = B // NW
CHUNK = 128
NCHUNK = BPW // CHUNK
LANES = 16


def _gmf_body(ut_hbm, it_hbm, ui_hbm, ii_hbm, out_hbm,
              ui_v, ii_v, u_buf, i_buf, o_buf,
              sem_g0, sem_g1, sem_o0, sem_o1, sem_ix):
    sem_g = (sem_g0, sem_g1)
    sem_o = (sem_o0, sem_o1)
    wid = lax.axis_index("s") * NC + lax.axis_index("c")
    base = wid * BPW

    cu = pltpu.async_copy(ui_hbm.at[wid], ui_v, sem_ix)
    ci = pltpu.async_copy(ii_hbm.at[wid], ii_v, sem_ix)
    cu.wait()
    ci.wait()

    def gathers(j, s):
        cu = pltpu.async_copy(ut_hbm.at[ui_v.at[j]], u_buf.at[s], sem_g[s])
        ci = pltpu.async_copy(it_hbm.at[ii_v.at[j]], i_buf.at[s], sem_g[s])
        return cu, ci

    pend_g = [gathers(0, 0), gathers(1, 1)]
    pend_o = [None, None]
    for j in range(NCHUNK):
        s = j % 2
        pend_g[s][0].wait()
        pend_g[s][1].wait()
        if pend_o[s] is not None:
            pend_o[s].wait()
            pend_o[s] = None

        def row_body(r, carry):
            for g in range(D // LANES):
                sl = pl.ds(g * LANES, LANES)
                o_buf[s, r, sl] = u_buf[s, r, sl] * i_buf[s, r, sl]
            return carry

        lax.fori_loop(0, CHUNK, row_body, 0)
        pend_o[s] = pltpu.async_copy(
            o_buf.at[s], out_hbm.at[pl.ds(base + j * CHUNK, CHUNK)], sem_o[s])
        if j + 2 < NCHUNK:
            pend_g[s] = gathers(j + 2, s)
    for s in range(2):
        if pend_o[s] is not None:
            pend_o[s].wait()


@functools.partial(jax.jit, static_argnames=())
def _gmf(user_table, item_table, user_indices, item_indices):
    mesh = plsc.VectorSubcoreMesh(core_axis_name="c", subcore_axis_name="s")
    call = pl.kernel(
        _gmf_body,
        mesh=mesh,
        out_type=jax.ShapeDtypeStruct((B, D), jnp.float32),
        scratch_types=[
            pltpu.VMEM((NCHUNK, CHUNK), jnp.int32),
            pltpu.VMEM((NCHUNK, CHUNK), jnp.int32),
            pltpu.VMEM((2, CHUNK, D), jnp.float32),
            pltpu.VMEM((2, CHUNK, D), jnp.float32),
            pltpu.VMEM((2, CHUNK, D), jnp.float32),
            pltpu.SemaphoreType.DMA,
            pltpu.SemaphoreType.DMA,
            pltpu.SemaphoreType.DMA,
            pltpu.SemaphoreType.DMA,
            pltpu.SemaphoreType.DMA,
        ],
    )
    return call(user_table, item_table, user_indices, item_indices)


def kernel(user_indices, item_indices, user_table, item_table):
    ui = user_indices.astype(jnp.int32).reshape(NW, NCHUNK, CHUNK)
    ii = item_indices.astype(jnp.int32).reshape(NW, NCHUNK, CHUNK)
    return _gmf(user_table, item_table, ui, ii)


# speedup vs baseline: 1.1291x; 1.0046x over previous
import functools

import jax
import jax.numpy as jnp
from jax import lax
from jax.experimental import pallas as pl
from jax.experimental.pallas import tpu as pltpu
from jax.experimental.pallas import tpu_sc as plsc

B = 16384
D = 128
NC = 2
NS = 16
NW = NC * NS
BPW = B // NW
CHUNK = 64
NCHUNK = BPW // CHUNK
NSETS = 4
NOSETS = 2
LANES = 16


def _gmf_body(ut_hbm, it_hbm, ui_hbm, ii_hbm, out_hbm,
              ui_v, ii_v, u_buf, i_buf, o_buf, *sems):
    sem_g = sems[:NSETS]
    sem_o = sems[NSETS:NSETS + NOSETS]
    sem_ix = sems[NSETS + NOSETS]
    wid = lax.axis_index("s") * NC + lax.axis_index("c")
    base = wid * BPW

    cu = pltpu.async_copy(ui_hbm.at[wid], ui_v, sem_ix)
    ci = pltpu.async_copy(ii_hbm.at[wid], ii_v, sem_ix)
    cu.wait()
    ci.wait()

    def gathers(j, s):
        cu = pltpu.async_copy(ut_hbm.at[ui_v.at[j]], u_buf.at[s], sem_g[s])
        ci = pltpu.async_copy(it_hbm.at[ii_v.at[j]], i_buf.at[s], sem_g[s])
        return cu, ci

    pend_g = [gathers(k, k) for k in range(NSETS)]
    pend_o = [None] * NOSETS
    for j in range(NCHUNK):
        s = j % NSETS
        so = j % NOSETS
        pend_g[s][0].wait()
        pend_g[s][1].wait()
        if pend_o[so] is not None:
            pend_o[so].wait()
            pend_o[so] = None

        def row_body(r, carry):
            for g in range(D // LANES):
                sl = pl.ds(g * LANES, LANES)
                o_buf[so, r, sl] = u_buf[s, r, sl] * i_buf[s, r, sl]
            return carry

        lax.fori_loop(0, CHUNK, row_body, 0)
        pend_o[so] = pltpu.async_copy(
            o_buf.at[so], out_hbm.at[pl.ds(base + j * CHUNK, CHUNK)], sem_o[so])
        if j + NSETS < NCHUNK:
            pend_g[s] = gathers(j + NSETS, s)
    for so in range(NOSETS):
        if pend_o[so] is not None:
            pend_o[so].wait()


@functools.partial(jax.jit, static_argnames=())
def _gmf(user_table, item_table, user_indices, item_indices):
    mesh = plsc.VectorSubcoreMesh(core_axis_name="c", subcore_axis_name="s")
    call = pl.kernel(
        _gmf_body,
        mesh=mesh,
        out_type=jax.ShapeDtypeStruct((B, D), jnp.float32),
        scratch_types=[
            pltpu.VMEM((NCHUNK, CHUNK), jnp.int32),
            pltpu.VMEM((NCHUNK, CHUNK), jnp.int32),
            pltpu.VMEM((NSETS, CHUNK, D), jnp.float32),
            pltpu.VMEM((NSETS, CHUNK, D), jnp.float32),
            pltpu.VMEM((NOSETS, CHUNK, D), jnp.float32),
        ] + [pltpu.SemaphoreType.DMA] * (NSETS + NOSETS + 1),
    )
    return call(user_table, item_table, user_indices, item_indices)


def kernel(user_indices, item_indices, user_table, item_table):
    ui = user_indices.astype(jnp.int32).reshape(NW, NCHUNK, CHUNK)
    ii = item_indices.astype(jnp.int32).reshape(NW, NCHUNK, CHUNK)
    return _gmf(user_table, item_table, ui, ii)


# speedup vs baseline: 1.1834x; 1.0481x over previous
import functools

import jax
import jax.numpy as jnp
from jax import lax
from jax.experimental import pallas as pl
from jax.experimental.pallas import tpu as pltpu
from jax.experimental.pallas import tpu_sc as plsc

B = 16384
D = 128
NC = 2
NS = 16
NW = NC * NS
BPW = B // NW
CHUNK = 64
NCHUNK = BPW // CHUNK
NSETS = 6
NOSETS = 3
LANES = 16


def _gmf_body(ut_hbm, it_hbm, ui_hbm, ii_hbm, out_hbm,
              ui_v, ii_v, u_buf, i_buf, o_buf, *sems):
    sem_g = sems[:NSETS]
    sem_o = sems[NSETS:NSETS + NOSETS]
    sem_ix = sems[NSETS + NOSETS]
    wid = lax.axis_index("s") * NC + lax.axis_index("c")
    base = wid * BPW

    cu = pltpu.async_copy(ui_hbm.at[wid], ui_v, sem_ix)
    ci = pltpu.async_copy(ii_hbm.at[wid], ii_v, sem_ix)
    cu.wait()
    ci.wait()

    def gathers(j, s):
        cu = pltpu.async_copy(ut_hbm.at[ui_v.at[j]], u_buf.at[s], sem_g[s])
        ci = pltpu.async_copy(it_hbm.at[ii_v.at[j]], i_buf.at[s], sem_g[s])
        return cu, ci

    pend_g = [gathers(k, k) for k in range(NSETS)]
    pend_o = [None] * NOSETS
    for j in range(NCHUNK):
        s = j % NSETS
        so = j % NOSETS
        pend_g[s][0].wait()
        pend_g[s][1].wait()
        if pend_o[so] is not None:
            pend_o[so].wait()
            pend_o[so] = None

        def row_body(r, carry):
            for g in range(D // LANES):
                sl = pl.ds(g * LANES, LANES)
                o_buf[so, r, sl] = u_buf[s, r, sl] * i_buf[s, r, sl]
            return carry

        lax.fori_loop(0, CHUNK, row_body, 0)
        pend_o[so] = pltpu.async_copy(
            o_buf.at[so], out_hbm.at[pl.ds(base + j * CHUNK, CHUNK)], sem_o[so])
        if j + NSETS < NCHUNK:
            pend_g[s] = gathers(j + NSETS, s)
    for so in range(NOSETS):
        if pend_o[so] is not None:
            pend_o[so].wait()


@functools.partial(jax.jit, static_argnames=())
def _gmf(user_table, item_table, user_indices, item_indices):
    mesh = plsc.VectorSubcoreMesh(core_axis_name="c", subcore_axis_name="s")
    call = pl.kernel(
        _gmf_body,
        mesh=mesh,
        out_type=jax.ShapeDtypeStruct((B, D), jnp.float32),
        scratch_types=[
            pltpu.VMEM((NCHUNK, CHUNK), jnp.int32),
            pltpu.VMEM((NCHUNK, CHUNK), jnp.int32),
            pltpu.VMEM((NSETS, CHUNK, D), jnp.float32),
            pltpu.VMEM((NSETS, CHUNK, D), jnp.float32),
            pltpu.VMEM((NOSETS, CHUNK, D), jnp.float32),
        ] + [pltpu.SemaphoreType.DMA] * (NSETS + NOSETS + 1),
    )
    return call(user_table, item_table, user_indices, item_indices)


def kernel(user_indices, item_indices, user_table, item_table):
    ui = user_indices.astype(jnp.int32).reshape(NW, NCHUNK, CHUNK)
    ii = item_indices.astype(jnp.int32).reshape(NW, NCHUNK, CHUNK)
    return _gmf(user_table, item_table, ui, ii)
